# baseline (speedup 1.0000x reference)
import functools

import numpy as np
import jax
import jax.numpy as jnp
from jax import lax
from jax.experimental import pallas as pl
from jax.experimental.pallas import tpu as pltpu
from jax.experimental.pallas import tpu_sc as plsc

_LANES = 16


def _pos_encoding(length, d_model, n=10000):
    d2 = d_model / 2
    position = np.arange(length)[:, np.newaxis]
    index = np.arange(int(d2))[np.newaxis, :]
    angle = position * np.power(n, -index / d2)
    return np.concatenate([np.sin(angle), np.cos(angle)], axis=-1).astype(np.float32)


def kernel(inputs, table):
    B, T = inputs.shape
    V, D = table.shape
    NW = 32
    seqs_per_w = B // NW
    n_lane = D // _LANES
    scale = float(np.sqrt(D))
    TP = 224
    G1 = 112
    G2 = TP - G1

    pe = jnp.asarray(_pos_encoding(T, D))
    idx_flat = inputs.reshape(-1).astype(jnp.int32)
    table2 = table.reshape(V // 2, 2 * D)

    mesh = plsc.VectorSubcoreMesh(core_axis_name="c", subcore_axis_name="s")

    @functools.partial(
        pl.kernel,
        mesh=mesh,
        out_type=jax.ShapeDtypeStruct((B, T, D), jnp.float32),
        scratch_types=[
            pltpu.VMEM((TP,), jnp.int32),
            pltpu.VMEM((TP,), jnp.int32),
            pltpu.VMEM((T, D), jnp.float32),
            pltpu.VMEM((TP, 2 * D), jnp.float32),
            pltpu.VMEM((T, D), jnp.float32),
            pltpu.SemaphoreType.DMA,
        ],
    )
    def emb(idx_hbm, table_hbm, pe_hbm, out_hbm, idx_v, idxg_v, pe_v, rows_v,
            out_v, sem):
        cid = lax.axis_index("c")
        sid = lax.axis_index("s")
        wid = sid * 2 + cid
        base_seq = wid * seqs_per_w
        pltpu.sync_copy(pe_hbm, pe_v)
        zeros = jnp.zeros((_LANES,), jnp.int32)
        idx_v[pl.ds(TP - 2 * _LANES, _LANES)] = zeros
        idx_v[pl.ds(TP - _LANES, _LANES)] = zeros

        def seq_body(s, carry):
            seq = base_seq + s
            pltpu.sync_copy(idx_hbm.at[pl.ds(seq * T, T)], idx_v.at[pl.ds(0, T)])
            for k in range(TP // _LANES):
                sl = pl.ds(k * _LANES, _LANES)
                idxg_v[sl] = jax.lax.shift_right_logical(idx_v[sl], 1)
            c1 = pltpu.async_copy(
                table_hbm.at[idxg_v.at[pl.ds(0, G1)]], rows_v.at[pl.ds(0, G1)], sem)
            c2 = pltpu.async_copy(
                table_hbm.at[idxg_v.at[pl.ds(G1, G2)]], rows_v.at[pl.ds(G1, G2)], sem)
            c1.wait()
            c2.wait()

            def row_body(t, c3):
                off = (idx_v[pl.ds(t, _LANES)][0] & 1) * D
                for l in range(n_lane):
                    sl = pl.ds(l * _LANES, _LANES)
                    out_v[t, sl] = (rows_v[t, pl.ds(off + l * _LANES, _LANES)]
                                    * scale + pe_v[t, sl])
                return c3

            lax.fori_loop(0, T, row_body, 0)
            pltpu.sync_copy(out_v, out_hbm.at[seq])
            return carry

        lax.fori_loop(0, seqs_per_w, seq_body, 0)

    return emb(idx_flat, table2, pe)


# speedup vs baseline: 1.6311x; 1.6311x over previous
import functools

import numpy as np
import jax
import jax.numpy as jnp
from jax import lax
from jax.experimental import pallas as pl
from jax.experimental.pallas import tpu as pltpu
from jax.experimental.pallas import tpu_sc as plsc

_LANES = 16


def _pos_encoding(length, d_model, n=10000):
    d2 = d_model / 2
    position = np.arange(length)[:, np.newaxis]
    index = np.arange(int(d2))[np.newaxis, :]
    angle = position * np.power(n, -index / d2)
    return np.concatenate([np.sin(angle), np.cos(angle)], axis=-1).astype(np.float32)


def kernel(inputs, table):
    B, T = inputs.shape
    V, D = table.shape
    NW = 32
    seqs_per_w = B // NW
    n_lane = D // _LANES
    scale = float(np.sqrt(D))
    TP = 208
    G1 = 112
    G2 = TP - G1

    pe = jnp.asarray(_pos_encoding(T, D))
    idx_flat = inputs.reshape(-1).astype(jnp.int32)
    table_p = jnp.pad(table, ((0, 0), (0, 2 * D - D)))

    mesh = plsc.VectorSubcoreMesh(core_axis_name="c", subcore_axis_name="s")

    @functools.partial(
        pl.kernel,
        mesh=mesh,
        out_type=jax.ShapeDtypeStruct((B, T, D), jnp.float32),
        scratch_types=[
            pltpu.VMEM((TP,), jnp.int32),
            pltpu.VMEM((T, D), jnp.float32),
            pltpu.VMEM((TP, 2 * D), jnp.float32),
            pltpu.VMEM((T, D), jnp.float32),
            pltpu.SemaphoreType.DMA,
        ],
    )
    def emb(idx_hbm, table_hbm, pe_hbm, out_hbm, idx_v, pe_v, rows_v, out_v, sem):
        cid = lax.axis_index("c")
        sid = lax.axis_index("s")
        wid = sid * 2 + cid
        base_seq = wid * seqs_per_w
        pltpu.sync_copy(pe_hbm, pe_v)
        zeros = jnp.zeros((_LANES,), jnp.int32)
        idx_v[pl.ds(TP - _LANES, _LANES)] = zeros

        def seq_body(s, carry):
            seq = base_seq + s
            pltpu.sync_copy(idx_hbm.at[pl.ds(seq * T, T)], idx_v.at[pl.ds(0, T)])
            c1 = pltpu.async_copy(
                table_hbm.at[idx_v.at[pl.ds(0, G1)]], rows_v.at[pl.ds(0, G1)], sem)
            c2 = pltpu.async_copy(
                table_hbm.at[idx_v.at[pl.ds(G1, G2)]], rows_v.at[pl.ds(G1, G2)], sem)
            c1.wait()
            c2.wait()

            def row_body(t, c3):
                for l in range(n_lane):
                    sl = pl.ds(l * _LANES, _LANES)
                    out_v[t, sl] = rows_v[t, sl] * scale + pe_v[t, sl]
                return c3

            lax.fori_loop(0, T, row_body, 0)
            pltpu.sync_copy(out_v, out_hbm.at[seq])
            return carry

        lax.fori_loop(0, seqs_per_w, seq_body, 0)

    return emb(idx_flat, table_p, pe)


# speedup vs baseline: 2.4924x; 1.5281x over previous
import functools

import numpy as np
import jax
import jax.numpy as jnp
from jax import lax
from jax.experimental import pallas as pl
from jax.experimental.pallas import tpu as pltpu
from jax.experimental.pallas import tpu_sc as plsc

_LANES = 16


def _pos_encoding(length, d_model, n=10000):
    d2 = d_model / 2
    position = np.arange(length)[:, np.newaxis]
    index = np.arange(int(d2))[np.newaxis, :]
    angle = position * np.power(n, -index / d2)
    return np.concatenate([np.sin(angle), np.cos(angle)], axis=-1).astype(np.float32)


def kernel(inputs, table):
    B, T = inputs.shape
    V, D = table.shape
    NW = 32
    SW = B // NW
    n_lane = D // _LANES
    scale = float(np.sqrt(D))
    TP = 224
    G1 = 112
    NI = SW * T + 2 * _LANES

    pe = jnp.asarray(_pos_encoding(T, D))
    idx_flat = inputs.reshape(-1).astype(jnp.int32)
    table_p = jnp.pad(table, ((0, 0), (0, D)))

    mesh = plsc.VectorSubcoreMesh(core_axis_name="c", subcore_axis_name="s")

    @functools.partial(
        pl.kernel,
        mesh=mesh,
        out_type=jax.ShapeDtypeStruct((B, T, D), jnp.float32),
        scratch_types=[
            pltpu.VMEM((NI,), jnp.int32),
            pltpu.VMEM((T, D), jnp.float32),
            pltpu.VMEM((TP, 2 * D), jnp.float32),
            pltpu.VMEM((TP, 2 * D), jnp.float32),
            pltpu.VMEM((T, D), jnp.float32),
            pltpu.SemaphoreType.DMA,
            pltpu.SemaphoreType.DMA,
        ],
    )
    def emb(idx_hbm, table_hbm, pe_hbm, out_hbm, idx_v, pe_v, rows0_v, rows1_v,
            out_v, sem0, sem1):
        cid = lax.axis_index("c")
        sid = lax.axis_index("s")
        wid = sid * 2 + cid
        base_seq = wid * SW
        zeros = jnp.zeros((_LANES,), jnp.int32)
        idx_v[pl.ds(NI - 2 * _LANES, _LANES)] = zeros
        idx_v[pl.ds(NI - _LANES, _LANES)] = zeros
        pltpu.sync_copy(idx_hbm.at[pl.ds(base_seq * T, SW * T)],
                        idx_v.at[pl.ds(0, SW * T)])
        pltpu.sync_copy(pe_hbm, pe_v)

        def gathers(s, rows_v, sem):
            for g in range(2):
                pltpu.async_copy(
                    table_hbm.at[idx_v.at[pl.ds(s * T + g * G1, G1)]],
                    rows_v.at[pl.ds(g * G1, G1)], sem)

        def drain(rows_v, sem):
            pltpu.make_async_copy(table_hbm.at[pl.ds(0, TP)], rows_v, sem).wait()

        def process(s, rows_v):
            def row_body(t, c3):
                for l in range(n_lane):
                    sl = pl.ds(l * _LANES, _LANES)
                    out_v[t, sl] = rows_v[t, sl] * scale + pe_v[t, sl]
                return c3

            lax.fori_loop(0, T, row_body, 0)
            pltpu.sync_copy(out_v, out_hbm.at[base_seq + s])

        gathers(0, rows0_v, sem0)

        def pair_body(s2, carry):
            s_even = 2 * s2
            gathers(s_even + 1, rows1_v, sem1)
            drain(rows0_v, sem0)
            process(s_even, rows0_v)

            @pl.when(s2 < SW // 2 - 1)
            def _():
                gathers(s_even + 2, rows0_v, sem0)

            drain(rows1_v, sem1)
            process(s_even + 1, rows1_v)
            return carry

        lax.fori_loop(0, SW // 2, pair_body, 0)

    return emb(idx_flat, table_p, pe)
